# baseline (speedup 1.0000x reference)
import jax
import jax.numpy as jnp
from jax import lax
from jax.experimental import pallas as pl
from jax.experimental.pallas import tpu as pltpu
from jax.experimental.pallas import tpu_sc as plsc

DIM = 64
LUT_SIZE = 65536
NUM_CORES = 2
NUM_SUBCORES = 16
NUM_WORKERS = NUM_CORES * NUM_SUBCORES
LANES = 16
BCHUNK = 128
NBUF = 2
SKEW = 65


def _sc_body(ids_hbm, lut_hbm, bf_hbm, out_hbm,
             ids_v, lut_sh, lut_v, bf_v, c_v, out_v,
             sem_l, sems_b, sems_o):
    hist = ids_hbm.shape[0]
    sid = lax.axis_index("s")
    wid = sid * NUM_CORES + lax.axis_index("c")
    b0 = wid * BCHUNK

    pltpu.sync_copy(ids_hbm.at[:, pl.ds(b0, BCHUNK)], ids_v)

    def issue(h, slot):
        pltpu.async_copy(bf_hbm.at[ids_v.at[h]], bf_v.at[slot], sems_b[slot])

    def wait_in(slot):
        pltpu.make_async_copy(
            bf_hbm.at[ids_v.at[0]], bf_v.at[slot], sems_b[slot]).wait()

    def wait_out(slot):
        pltpu.make_async_copy(
            out_v.at[slot],
            out_hbm.at[0, :, 0, :], sems_o[slot]).wait()

    issue(0, 0)

    with jax.named_scope("lut_stage"):
        @pl.when(sid == 0)
        def _():
            pltpu.sync_copy(lut_hbm, lut_sh)
        plsc.subcore_barrier()
        pltpu.async_copy(lut_sh, lut_v, sem_l).wait()

    HALF = BCHUNK // 2
    t65 = [(jax.lax.iota(jnp.int32, 16) + tb * LANES) * SKEW
           for tb in range(HALF // LANES)]

    def outer(g, carry):
        for b in range(NBUF):
            cur = g * NBUF + b
            nxt = cur + 1

            @pl.when(nxt < hist)
            def _():
                issue(nxt, (b + 1) % NBUF)

            wait_in(b)

            @pl.when(cur >= NBUF)
            def _():
                wait_out(b)

            for half in range(2):
                t_off = half * HALF

                @plsc.parallel_loop(0, HALF, 1, unroll=8)
                def _(t):
                    row = t * SKEW
                    for j in range(DIM // LANES):
                        c_v[pl.ds(row + j * LANES, LANES)] = (
                            bf_v[b, t_off + t, pl.ds(j * LANES, LANES)] * 256
                            + bf_v[b, t_off + t,
                                   pl.ds(DIM + j * LANES, LANES)])

                @plsc.parallel_loop(0, DIM, 1, unroll=4)
                def _(d):
                    dhi = d // 8
                    off = (d % 8) * BCHUNK + t_off
                    d_vec = jnp.full((16,), d, dtype=jnp.int32)
                    for tb in range(HALF // LANES):
                        cc = plsc.load_gather(c_v, [t65[tb] + d_vec])
                        out_v[b, dhi, pl.ds(off + tb * LANES, LANES)] = (
                            plsc.load_gather(lut_v, [cc]))

            pltpu.async_copy(
                out_v.at[b],
                out_hbm.at[cur, :, wid, :],
                sems_o[b])
        return carry

    with jax.named_scope("main_loop"):
        lax.fori_loop(0, hist // NBUF, outer, 0)
    for b in range(NBUF):
        wait_out(b)


@jax.jit
def _sc_embed(ids_hb, lut, bf):
    hist, batch = ids_hb.shape
    mesh = plsc.VectorSubcoreMesh(core_axis_name="c", subcore_axis_name="s")
    n_btiles = batch // BCHUNK
    kern = pl.kernel(
        _sc_body,
        out_type=jax.ShapeDtypeStruct((hist, DIM // 8, n_btiles, 8 * BCHUNK),
                                      jnp.float32),
        mesh=mesh,
        scratch_types=[
            pltpu.VMEM((hist, BCHUNK), jnp.int32),
            pltpu.MemorySpace.VMEM_SHARED((LUT_SIZE,), jnp.float32),
            pltpu.VMEM((LUT_SIZE,), jnp.float32),
            pltpu.VMEM((NBUF, BCHUNK, 2 * DIM), jnp.int32),
            pltpu.VMEM((BCHUNK // 2 * SKEW,), jnp.int32),
            pltpu.VMEM((NBUF, DIM // 8, 8 * BCHUNK), jnp.float32),
            pltpu.SemaphoreType.DMA,
            [pltpu.SemaphoreType.DMA] * NBUF,
            [pltpu.SemaphoreType.DMA] * NBUF,
        ],
        compiler_params=pltpu.CompilerParams(
            needs_layout_passes=False, use_tc_tiling_on_sc=False),
    )
    return kern(ids_hb, lut, bf)


def kernel(input_ids, lut, base_idx, fine_idx):
    ids_hb = jnp.transpose(input_ids.astype(jnp.int32), (1, 0))
    bf = lax.pad(base_idx.astype(jnp.int32), jnp.int32(0),
                 ((0, 0, 0), (0, DIM, 0)))
    bf = lax.dynamic_update_slice(bf, fine_idx.astype(jnp.int32), (0, DIM))
    out4 = _sc_embed(ids_hb, lut, bf)
    hist, batch = ids_hb.shape
    x = out4.reshape(hist, DIM // 8, batch // BCHUNK, 8, BCHUNK)
    return jnp.transpose(x, (2, 4, 0, 1, 3)).reshape(batch, hist, DIM)


# speedup vs baseline: 2.4066x; 2.4066x over previous
import jax
import jax.numpy as jnp
from jax import lax
from jax.experimental import pallas as pl
from jax.experimental.pallas import tpu as pltpu
from jax.experimental.pallas import tpu_sc as plsc

DIM = 64
LUT_SIZE = 65536
NUM_CORES = 2
NUM_SUBCORES = 16
NUM_WORKERS = NUM_CORES * NUM_SUBCORES
LANES = 16
BCHUNK = 128
NBUF = 2
SKEW = 65


def _sc_body(ids_hbm, lut_hbm, bf_hbm, out_hbm,
             ids_v, lut_sh, lut_v, bf_v, c_v, out_v,
             sem_l, sems_b, sems_o):
    hist = ids_hbm.shape[0]
    sid = lax.axis_index("s")
    wid = sid * NUM_CORES + lax.axis_index("c")
    b0 = wid * BCHUNK

    pltpu.sync_copy(ids_hbm.at[:, pl.ds(b0, BCHUNK)], ids_v)

    def issue(h, slot):
        pltpu.async_copy(bf_hbm.at[ids_v.at[h]], bf_v.at[slot], sems_b[slot])

    def wait_in(slot):
        pltpu.make_async_copy(
            bf_hbm.at[ids_v.at[0]], bf_v.at[slot], sems_b[slot]).wait()

    def wait_out(slot):
        pltpu.make_async_copy(
            out_v.at[slot],
            out_hbm.at[0, :, 0, :], sems_o[slot]).wait()

    issue(0, 0)

    with jax.named_scope("lut_stage"):
        @pl.when(sid == 0)
        def _():
            pltpu.sync_copy(lut_hbm, lut_sh)
        plsc.subcore_barrier()
        pltpu.async_copy(lut_sh, lut_v, sem_l).wait()

    HALF = BCHUNK // 2
    t65 = [(jax.lax.iota(jnp.int32, 16) + tb * LANES) * SKEW
           for tb in range(HALF // LANES)]

    def outer(g, carry):
        for b in range(NBUF):
            cur = g * NBUF + b
            nxt = cur + 1

            @pl.when(nxt < hist)
            def _():
                issue(nxt, (b + 1) % NBUF)

            wait_in(b)

            @pl.when(cur >= NBUF)
            def _():
                wait_out(b)

            for half in range(2):
                t_off = half * HALF

                @plsc.parallel_loop(0, HALF, 1, unroll=8)
                def _(t):
                    row = t * SKEW
                    for j in range(DIM // LANES):
                        c_v[pl.ds(row + j * LANES, LANES)] = (
                            bf_v[b, t_off + t, pl.ds(j * LANES, LANES)] * 256
                            + bf_v[b, t_off + t,
                                   pl.ds(DIM + j * LANES, LANES)])

                @plsc.parallel_loop(0, DIM, 1, unroll=4)
                def _(d):
                    dhi = d // 8
                    off = (d % 8) * BCHUNK + t_off
                    d_vec = jnp.full((16,), d, dtype=jnp.int32)
                    for tb in range(HALF // LANES):
                        cc = plsc.load_gather(c_v, [t65[tb] + d_vec])
                        out_v[b, dhi, pl.ds(off + tb * LANES, LANES)] = (
                            plsc.load_gather(lut_v, [cc]))

            pltpu.async_copy(
                out_v.at[b],
                out_hbm.at[cur, :, wid, :],
                sems_o[b])
        return carry

    with jax.named_scope("main_loop"):
        lax.fori_loop(0, hist // NBUF, outer, 0)
    for b in range(NBUF):
        wait_out(b)


@jax.jit
def _sc_embed(ids_hb, lut, bf):
    hist, batch = ids_hb.shape
    mesh = plsc.VectorSubcoreMesh(core_axis_name="c", subcore_axis_name="s")
    n_btiles = batch // BCHUNK
    kern = pl.kernel(
        _sc_body,
        out_type=jax.ShapeDtypeStruct((hist, DIM // 8, n_btiles, 8 * BCHUNK),
                                      jnp.float32),
        mesh=mesh,
        scratch_types=[
            pltpu.VMEM((hist, BCHUNK), jnp.int32),
            pltpu.MemorySpace.VMEM_SHARED((LUT_SIZE,), jnp.float32),
            pltpu.VMEM((LUT_SIZE,), jnp.float32),
            pltpu.VMEM((NBUF, BCHUNK, 2 * DIM), jnp.int32),
            pltpu.VMEM((BCHUNK // 2 * SKEW,), jnp.int32),
            pltpu.VMEM((NBUF, DIM // 8, 8 * BCHUNK), jnp.float32),
            pltpu.SemaphoreType.DMA,
            [pltpu.SemaphoreType.DMA] * NBUF,
            [pltpu.SemaphoreType.DMA] * NBUF,
        ],
        compiler_params=pltpu.CompilerParams(
            needs_layout_passes=False, use_tc_tiling_on_sc=False),
    )
    return kern(ids_hb, lut, bf)


def kernel(input_ids, lut, base_idx, fine_idx):
    ids_hb = jnp.transpose(input_ids.astype(jnp.int32), (1, 0))
    bf = jnp.concatenate(
        [base_idx.astype(jnp.int32), fine_idx.astype(jnp.int32)], axis=1)
    out4 = _sc_embed(ids_hb, lut, bf)
    hist, batch = ids_hb.shape
    x = out4.reshape(hist, DIM // 8, batch // BCHUNK, 8, BCHUNK)
    return jnp.transpose(x, (2, 4, 0, 1, 3)).reshape(batch, hist, DIM)


# speedup vs baseline: 2.4333x; 1.0111x over previous
import jax
import jax.numpy as jnp
from jax import lax
from jax.experimental import pallas as pl
from jax.experimental.pallas import tpu as pltpu
from jax.experimental.pallas import tpu_sc as plsc

DIM = 64
LUT_SIZE = 65536
NUM_CORES = 2
NUM_SUBCORES = 16
NUM_WORKERS = NUM_CORES * NUM_SUBCORES
LANES = 16
BCHUNK = 128
NBUF = 2
SKEW = 33


def _sc_body(ids_hbm, lut_hbm, bf_hbm, out_hbm,
             ids_v, lut_sh, lut_v, bf_v, c_v, out_v,
             sem_l, sems_b, sems_o):
    hist = ids_hbm.shape[0]
    sid = lax.axis_index("s")
    wid = sid * NUM_CORES + lax.axis_index("c")
    b0 = wid * BCHUNK

    pltpu.sync_copy(ids_hbm.at[:, pl.ds(b0, BCHUNK)], ids_v)

    def issue(h, slot):
        pltpu.async_copy(bf_hbm.at[ids_v.at[h]], bf_v.at[slot], sems_b[slot])

    def wait_in(slot):
        pltpu.make_async_copy(
            bf_hbm.at[ids_v.at[0]], bf_v.at[slot], sems_b[slot]).wait()

    def wait_out(slot):
        pltpu.make_async_copy(
            out_v.at[slot],
            out_hbm.at[0, :, 0, :], sems_o[slot]).wait()

    issue(0, 0)

    with jax.named_scope("lut_stage"):
        @pl.when(sid == 0)
        def _():
            pltpu.sync_copy(lut_hbm, lut_sh)
        plsc.subcore_barrier()
        pltpu.async_copy(lut_sh, lut_v, sem_l).wait()

    HALF = BCHUNK // 2
    tskew = [(jax.lax.iota(jnp.int32, 16) + tb * LANES) * SKEW
             for tb in range(HALF // LANES)]

    def outer(g, carry):
        for b in range(NBUF):
            cur = g * NBUF + b
            nxt = cur + 1

            @pl.when(nxt < hist)
            def _():
                issue(nxt, (b + 1) % NBUF)

            wait_in(b)

            @pl.when(cur >= NBUF)
            def _():
                wait_out(b)

            for half in range(2):
                t_off = half * HALF

                @plsc.parallel_loop(0, HALF, 1, unroll=8)
                def _(t):
                    row = t * SKEW
                    c = [bf_v[b, t_off + t, pl.ds(j * LANES, LANES)] * 256
                         + bf_v[b, t_off + t, pl.ds(DIM + j * LANES, LANES)]
                         for j in range(DIM // LANES)]
                    c_v[pl.ds(row, LANES)] = c[0] | (c[1] << 16)
                    c_v[pl.ds(row + LANES, LANES)] = c[2] | (c[3] << 16)

                @plsc.parallel_loop(0, DIM // 2, 1, unroll=4)
                def _(p):
                    d_lo = p + (p & 16)
                    off_lo = (d_lo % 8) * BCHUNK + t_off
                    off_hi = ((d_lo + 16) % 8) * BCHUNK + t_off
                    p_vec = jnp.full((16,), p, dtype=jnp.int32)
                    for tb in range(HALF // LANES):
                        w = plsc.load_gather(c_v, [tskew[tb] + p_vec])
                        lo = w & 0xFFFF
                        hi = lax.shift_right_logical(w, 16)
                        out_v[b, d_lo // 8,
                              pl.ds(off_lo + tb * LANES, LANES)] = (
                            plsc.load_gather(lut_v, [lo]))
                        out_v[b, (d_lo + 16) // 8,
                              pl.ds(off_hi + tb * LANES, LANES)] = (
                            plsc.load_gather(lut_v, [hi]))

            pltpu.async_copy(
                out_v.at[b],
                out_hbm.at[cur, :, wid, :],
                sems_o[b])
        return carry

    with jax.named_scope("main_loop"):
        lax.fori_loop(0, hist // NBUF, outer, 0)
    for b in range(NBUF):
        wait_out(b)


@jax.jit
def _sc_embed(ids_hb, lut, bf):
    hist, batch = ids_hb.shape
    mesh = plsc.VectorSubcoreMesh(core_axis_name="c", subcore_axis_name="s")
    n_btiles = batch // BCHUNK
    kern = pl.kernel(
        _sc_body,
        out_type=jax.ShapeDtypeStruct((hist, DIM // 8, n_btiles, 8 * BCHUNK),
                                      jnp.float32),
        mesh=mesh,
        scratch_types=[
            pltpu.VMEM((hist, BCHUNK), jnp.int32),
            pltpu.MemorySpace.VMEM_SHARED((LUT_SIZE,), jnp.float32),
            pltpu.VMEM((LUT_SIZE,), jnp.float32),
            pltpu.VMEM((NBUF, BCHUNK, 2 * DIM), jnp.int32),
            pltpu.VMEM((BCHUNK // 2 * SKEW,), jnp.int32),
            pltpu.VMEM((NBUF, DIM // 8, 8 * BCHUNK), jnp.float32),
            pltpu.SemaphoreType.DMA,
            [pltpu.SemaphoreType.DMA] * NBUF,
            [pltpu.SemaphoreType.DMA] * NBUF,
        ],
        compiler_params=pltpu.CompilerParams(
            needs_layout_passes=False, use_tc_tiling_on_sc=False),
    )
    return kern(ids_hb, lut, bf)


def kernel(input_ids, lut, base_idx, fine_idx):
    ids_hb = jnp.transpose(input_ids.astype(jnp.int32), (1, 0))
    bf = jnp.concatenate(
        [base_idx.astype(jnp.int32), fine_idx.astype(jnp.int32)], axis=1)
    out4 = _sc_embed(ids_hb, lut, bf)
    hist, batch = ids_hb.shape
    x = out4.reshape(hist, DIM // 8, batch // BCHUNK, 8, BCHUNK)
    return jnp.transpose(x, (2, 4, 0, 1, 3)).reshape(batch, hist, DIM)
